# baseline (speedup 1.0000x reference)
import jax
import jax.numpy as jnp
from jax import lax
from jax.experimental import pallas as pl
from jax.experimental.pallas import tpu as pltpu
from jax.experimental.pallas import tpu_sc as plsc

N = 10000
M = 5000
NNZ = 160000
C = 256
H = 128
MP = 5120
NP = 10240
NSUB = 16
W = 128
NNZP = 163840
CHUNKP = NNZP // NSUB
NWINP = CHUNKP // W
BIG = 632
LAST = N - (NSUB - 1) * BIG

_mesh = lambda: plsc.VectorSubcoreMesh(core_axis_name="c", subcore_axis_name="s")


def _k1_body(x_ref, w_ref, b_ref, o_ref):
    o_ref[0] = (
        jax.lax.dot_general(
            x_ref[...], w_ref[...], (((1,), (1,)), ((), ())),
            preferred_element_type=jnp.float32,
            precision=jax.lax.Precision.HIGHEST,
        )
        + b_ref[0]
    )


def _k1(Xp, W_theta, b2):
    R = 1024
    return pl.pallas_call(
        _k1_body,
        grid=(NP // R, 2),
        in_specs=[
            pl.BlockSpec((R, C), lambda i, h: (i, 0)),
            pl.BlockSpec((H, C), lambda i, h: (h, 0)),
            pl.BlockSpec((1, 1, H), lambda i, h: (h, 0, 0)),
        ],
        out_specs=pl.BlockSpec((1, R, H), lambda i, h: (h, i, 0)),
        out_shape=jax.ShapeDtypeStruct((2, NP, H), jnp.float32),
    )(Xp, W_theta, b2)


def _k2_body(xw_hbm, vi_hbm, ei_hbm, ysum_hbm, cnt_hbm,
             ysum_sp, cnt_sp, vbuf, ebuf, rows, ones_b, zvec, sem_i):
    cid = lax.axis_index("c")
    sid = lax.axis_index("s")
    zr = MP // NSUB
    base = sid * CHUNKP

    z16 = jnp.zeros((16,), jnp.float32)
    o16 = jnp.ones((16,), jnp.float32)

    def zr_body(i, carry):
        for k in range(H // 16):
            rows[i, pl.ds(k * 16, 16)] = z16
        return carry

    lax.fori_loop(0, W, zr_body, 0)
    for k in range(W // 16):
        ones_b[pl.ds(k * 16, 16)] = o16
    for k in range(zr // 16):
        zvec[pl.ds(k * 16, 16)] = z16

    pltpu.sync_copy(rows, ysum_sp.at[pl.ds(sid * zr, W)])
    pltpu.sync_copy(rows, ysum_sp.at[pl.ds(sid * zr + W, W)])
    pltpu.sync_copy(rows.at[pl.ds(0, zr - 2 * W)],
                    ysum_sp.at[pl.ds(sid * zr + 2 * W, zr - 2 * W)])
    pltpu.sync_copy(zvec, cnt_sp.at[pl.ds(sid * zr, zr)])

    descs = []
    for j in range(NWINP):
        descs.append(pltpu.async_copy(
            vi_hbm.at[pl.ds(base + j * W, W)], vbuf.at[j], sem_i))
        descs.append(pltpu.async_copy(
            ei_hbm.at[pl.ds(base + j * W, W)], ebuf.at[j], sem_i))
    for d in descs:
        d.wait()
    plsc.subcore_barrier()

    def win(j, carry):
        pltpu.sync_copy(xw_hbm.at[cid].at[vbuf.at[j]], rows)
        pltpu.sync_copy(rows, ysum_sp.at[ebuf.at[j]], add=True)
        pltpu.sync_copy(ones_b, cnt_sp.at[ebuf.at[j]], add=True)
        return carry

    lax.fori_loop(0, NWINP, win, 0)
    plsc.subcore_barrier()

    pltpu.sync_copy(ysum_sp.at[pl.ds(sid * zr, zr)],
                    ysum_hbm.at[cid, pl.ds(sid * zr, zr)])

    @pl.when(cid == 0)
    def _():
        pltpu.sync_copy(cnt_sp.at[pl.ds(sid * zr, zr)], zvec)
        pltpu.sync_copy(zvec, cnt_hbm.at[pl.ds(sid * zr, zr)])


def _k2(Xw2, vp, ep):
    f = pl.kernel(
        _k2_body,
        out_type=(
            jax.ShapeDtypeStruct((2, MP, H), jnp.float32),
            jax.ShapeDtypeStruct((MP,), jnp.float32),
        ),
        mesh=_mesh(),
        scratch_types=[
            pltpu.VMEM_SHARED((MP, H), jnp.float32),
            pltpu.VMEM_SHARED((MP,), jnp.float32),
            pltpu.VMEM((NWINP, W), jnp.int32),
            pltpu.VMEM((NWINP, W), jnp.int32),
            pltpu.VMEM((W, H), jnp.float32),
            pltpu.VMEM((W,), jnp.float32),
            pltpu.VMEM((MP // NSUB,), jnp.float32),
            pltpu.SemaphoreType.DMA,
        ],
    )
    return f(Xw2, vp, ep)


def _k3_body(ysum_ref, cnt_ref, w_ref, ys_ref, ge_ref):
    c = jnp.maximum(cnt_ref[...], 1.0)
    y0 = ysum_ref[0] / c
    y1 = ysum_ref[1] / c
    a = jnp.sum(y0 * w_ref[0] + y1 * w_ref[1], axis=1, keepdims=True)
    a = jnp.where(a >= 0, a, 0.2 * a)
    a = jnp.clip(a, 0.001, 5.0)
    g = jnp.exp(a)
    ys_ref[0] = g * y0
    ys_ref[1] = g * y1
    ge_ref[...] = g


def _k3(ysum2, cnt2, w2):
    R = 640
    return pl.pallas_call(
        _k3_body,
        grid=(MP // R,),
        in_specs=[
            pl.BlockSpec((2, R, H), lambda i: (0, i, 0)),
            pl.BlockSpec((R, 1), lambda i: (i, 0)),
            pl.BlockSpec((2, 1, H), lambda i: (0, 0, 0)),
        ],
        out_specs=[
            pl.BlockSpec((2, R, H), lambda i: (0, i, 0)),
            pl.BlockSpec((R, 1), lambda i: (i, 0)),
        ],
        out_shape=[
            jax.ShapeDtypeStruct((2, MP, H), jnp.float32),
            jax.ShapeDtypeStruct((MP, 1), jnp.float32),
        ],
    )(ysum2, cnt2, w2)


def _k4_body(ys_hbm, ge_hbm, vi_hbm, ei_hbm, osum_hbm, den_hbm,
             oacc_sp, den_sp, vbuf, ebuf, rows, vals, zvec, sem_i):
    cid = lax.axis_index("c")
    sid = lax.axis_index("s")
    zr = NP // NSUB
    base = sid * CHUNKP

    z16 = jnp.zeros((16,), jnp.float32)

    def zr_body(i, carry):
        for k in range(H // 16):
            rows[i, pl.ds(k * 16, 16)] = z16
        return carry

    lax.fori_loop(0, W, zr_body, 0)
    for k in range(zr // 16):
        zvec[pl.ds(k * 16, 16)] = z16

    for b in range(zr // W):
        pltpu.sync_copy(rows, oacc_sp.at[pl.ds(sid * zr + b * W, W)])
    pltpu.sync_copy(zvec, den_sp.at[pl.ds(sid * zr, zr)])

    descs = []
    for j in range(NWINP):
        descs.append(pltpu.async_copy(
            vi_hbm.at[pl.ds(base + j * W, W)], vbuf.at[j], sem_i))
        descs.append(pltpu.async_copy(
            ei_hbm.at[pl.ds(base + j * W, W)], ebuf.at[j], sem_i))
    for d in descs:
        d.wait()
    plsc.subcore_barrier()

    def win(j, carry):
        pltpu.sync_copy(ys_hbm.at[cid].at[ebuf.at[j]], rows)
        pltpu.sync_copy(ge_hbm.at[ebuf.at[j]], vals)
        pltpu.sync_copy(rows, oacc_sp.at[vbuf.at[j]], add=True)
        pltpu.sync_copy(vals, den_sp.at[vbuf.at[j]], add=True)
        return carry

    lax.fori_loop(0, NWINP, win, 0)
    plsc.subcore_barrier()

    @pl.when(sid < NSUB - 1)
    def _():
        pltpu.sync_copy(oacc_sp.at[pl.ds(sid * BIG, BIG)],
                        osum_hbm.at[cid, pl.ds(sid * BIG, BIG)])

    @pl.when(sid == NSUB - 1)
    def _():
        pltpu.sync_copy(oacc_sp.at[pl.ds((NSUB - 1) * BIG, LAST)],
                        osum_hbm.at[cid, pl.ds((NSUB - 1) * BIG, LAST)])

    @pl.when(cid == 0)
    def _():
        pltpu.sync_copy(den_sp.at[pl.ds(sid * zr, zr)], zvec)
        pltpu.sync_copy(zvec, den_hbm.at[pl.ds(sid * zr, zr)])


def _k4(yscaled2, ge, vp, ep):
    f = pl.kernel(
        _k4_body,
        out_type=(
            jax.ShapeDtypeStruct((2, N, H), jnp.float32),
            jax.ShapeDtypeStruct((NP,), jnp.float32),
        ),
        mesh=_mesh(),
        scratch_types=[
            pltpu.VMEM_SHARED((NP, H), jnp.float32),
            pltpu.VMEM_SHARED((NP,), jnp.float32),
            pltpu.VMEM((NWINP, W), jnp.int32),
            pltpu.VMEM((NWINP, W), jnp.int32),
            pltpu.VMEM((W, H), jnp.float32),
            pltpu.VMEM((W,), jnp.float32),
            pltpu.VMEM((NP // NSUB,), jnp.float32),
            pltpu.SemaphoreType.DMA,
        ],
    )
    return f(yscaled2, ge, vp, ep)


def _k5_body(os_ref, den_ref, out_ref):
    d = jnp.maximum(den_ref[...], 1e-12)
    o = jnp.concatenate([os_ref[0], os_ref[1]], axis=1) / d
    out_ref[...] = jnp.where(o > 0, o, jnp.exp(o) - 1.0)


def _k5(outsum2, den2):
    R = 1000
    return pl.pallas_call(
        _k5_body,
        grid=(N // R,),
        in_specs=[
            pl.BlockSpec((2, R, H), lambda i: (0, i, 0)),
            pl.BlockSpec((R, 1), lambda i: (i, 0)),
        ],
        out_specs=pl.BlockSpec((R, C), lambda i: (i, 0)),
        out_shape=jax.ShapeDtypeStruct((N, C), jnp.float32),
    )(outsum2, den2)


def kernel(X, v_idx, e_idx, W_theta, b_theta, w_atten_e):
    b2 = b_theta.reshape(2, 1, H)
    w2 = w_atten_e.reshape(2, 1, H)
    Xp = jnp.pad(X, ((0, NP - N), (0, 0)))
    vp = jnp.pad(v_idx, (0, NNZP - NNZ), constant_values=N)
    ep = jnp.pad(e_idx, (0, NNZP - NNZ), constant_values=M)
    Xw2 = _k1(Xp, W_theta, b2)
    ysum2, cnt = _k2(Xw2, vp, ep)
    yscaled2, ge = _k3(ysum2, cnt.reshape(MP, 1), w2)
    outsum2, den = _k4(yscaled2, ge.reshape(MP), vp, ep)
    return _k5(outsum2, den[:N].reshape(N, 1))


# speedup vs baseline: 1.6403x; 1.6403x over previous
import jax
import jax.numpy as jnp
from jax import lax
from jax.experimental import pallas as pl
from jax.experimental.pallas import tpu as pltpu
from jax.experimental.pallas import tpu_sc as plsc

N = 10000
M = 5000
NNZ = 160000
C = 256
H = 128
MP = 5120
NP = 10240
NSUB = 16
W = 128
NNZP = 163840
CHUNKP = NNZP // NSUB
NWINP = CHUNKP // W
BIG = 632
LAST = N - (NSUB - 1) * BIG

_mesh = lambda: plsc.VectorSubcoreMesh(core_axis_name="c", subcore_axis_name="s")


def _k1_body(x_ref, w_ref, b_ref, o_ref):
    o_ref[0] = (
        jax.lax.dot_general(
            x_ref[...], w_ref[...], (((1,), (1,)), ((), ())),
            preferred_element_type=jnp.float32,
            precision=jax.lax.Precision.HIGHEST,
        )
        + b_ref[0]
    )


def _k1(Xp, W_theta, b2):
    R = 1024
    return pl.pallas_call(
        _k1_body,
        grid=(NP // R, 2),
        in_specs=[
            pl.BlockSpec((R, C), lambda i, h: (i, 0)),
            pl.BlockSpec((H, C), lambda i, h: (h, 0)),
            pl.BlockSpec((1, 1, H), lambda i, h: (h, 0, 0)),
        ],
        out_specs=pl.BlockSpec((1, R, H), lambda i, h: (h, i, 0)),
        out_shape=jax.ShapeDtypeStruct((2, NP, H), jnp.float32),
    )(Xp, W_theta, b2)


def _k2_body(xw_hbm, vi_hbm, ei_hbm, ysum_hbm, cnt_hbm,
             ysum_sp, cnt_sp, vbuf, ebuf, rows, ones_b, zvec, sem_i):
    cid = lax.axis_index("c")
    sid = lax.axis_index("s")
    zr = MP // NSUB
    base = sid * CHUNKP

    z16 = jnp.zeros((16,), jnp.float32)
    o16 = jnp.ones((16,), jnp.float32)

    def zr_body(i, carry):
        for k in range(H // 16):
            rows[i, pl.ds(k * 16, 16)] = z16
        return carry

    lax.fori_loop(0, W, zr_body, 0)
    for k in range(W // 16):
        ones_b[pl.ds(k * 16, 16)] = o16
    for k in range(zr // 16):
        zvec[pl.ds(k * 16, 16)] = z16

    pltpu.sync_copy(rows, ysum_sp.at[pl.ds(sid * zr, W)])
    pltpu.sync_copy(rows, ysum_sp.at[pl.ds(sid * zr + W, W)])
    pltpu.sync_copy(rows.at[pl.ds(0, zr - 2 * W)],
                    ysum_sp.at[pl.ds(sid * zr + 2 * W, zr - 2 * W)])
    pltpu.sync_copy(zvec, cnt_sp.at[pl.ds(sid * zr, zr)])

    descs = []
    for j in range(NWINP):
        descs.append(pltpu.async_copy(
            vi_hbm.at[pl.ds(base + j * W, W)], vbuf.at[j], sem_i))
        descs.append(pltpu.async_copy(
            ei_hbm.at[pl.ds(base + j * W, W)], ebuf.at[j], sem_i))
    for d in descs:
        d.wait()
    plsc.subcore_barrier()

    def win(j, carry):
        pltpu.sync_copy(xw_hbm.at[cid].at[vbuf.at[j]], rows)
        pltpu.sync_copy(rows, ysum_sp.at[ebuf.at[j]], add=True)
        pltpu.sync_copy(ones_b, cnt_sp.at[ebuf.at[j]], add=True)
        return carry

    lax.fori_loop(0, NWINP, win, 0)
    plsc.subcore_barrier()

    pltpu.sync_copy(ysum_sp.at[pl.ds(sid * zr, zr)],
                    ysum_hbm.at[cid, pl.ds(sid * zr, zr)])

    @pl.when(cid == 0)
    def _():
        pltpu.sync_copy(cnt_sp.at[pl.ds(sid * zr, zr)], zvec)
        pltpu.sync_copy(zvec, cnt_hbm.at[pl.ds(sid * zr, zr)])


def _k2(Xw2, vp, ep):
    f = pl.kernel(
        _k2_body,
        out_type=(
            jax.ShapeDtypeStruct((2, MP, H), jnp.float32),
            jax.ShapeDtypeStruct((MP,), jnp.float32),
        ),
        mesh=_mesh(),
        scratch_types=[
            pltpu.VMEM_SHARED((MP, H), jnp.float32),
            pltpu.VMEM_SHARED((MP,), jnp.float32),
            pltpu.VMEM((NWINP, W), jnp.int32),
            pltpu.VMEM((NWINP, W), jnp.int32),
            pltpu.VMEM((W, H), jnp.float32),
            pltpu.VMEM((W,), jnp.float32),
            pltpu.VMEM((MP // NSUB,), jnp.float32),
            pltpu.SemaphoreType.DMA,
        ],
    )
    return f(Xw2, vp, ep)


def _k3_body(ysum_ref, cnt_ref, w_ref, ys_ref, ge_ref):
    c = jnp.maximum(cnt_ref[...], 1.0)
    y0 = ysum_ref[0] / c
    y1 = ysum_ref[1] / c
    a = jnp.sum(y0 * w_ref[0] + y1 * w_ref[1], axis=1, keepdims=True)
    a = jnp.where(a >= 0, a, 0.2 * a)
    a = jnp.clip(a, 0.001, 5.0)
    g = jnp.exp(a)
    ys_ref[0] = g * y0
    ys_ref[1] = g * y1
    ge_ref[...] = g


def _k3(ysum2, cnt2, w2):
    R = 640
    return pl.pallas_call(
        _k3_body,
        grid=(MP // R,),
        in_specs=[
            pl.BlockSpec((2, R, H), lambda i: (0, i, 0)),
            pl.BlockSpec((R, 1), lambda i: (i, 0)),
            pl.BlockSpec((2, 1, H), lambda i: (0, 0, 0)),
        ],
        out_specs=[
            pl.BlockSpec((2, R, H), lambda i: (0, i, 0)),
            pl.BlockSpec((R, 1), lambda i: (i, 0)),
        ],
        out_shape=[
            jax.ShapeDtypeStruct((2, MP, H), jnp.float32),
            jax.ShapeDtypeStruct((MP, 1), jnp.float32),
        ],
    )(ysum2, cnt2, w2)


def _k4_body(ys_hbm, ge_hbm, vi_hbm, ei_hbm, osum_hbm, den_hbm,
             oacc_sp, den_sp, vbuf, ebuf, rows, vals, zvec, sem_i):
    cid = lax.axis_index("c")
    sid = lax.axis_index("s")
    zr = NP // NSUB
    base = sid * CHUNKP

    z16 = jnp.zeros((16,), jnp.float32)

    def zr_body(i, carry):
        for k in range(H // 16):
            rows[i, pl.ds(k * 16, 16)] = z16
        return carry

    lax.fori_loop(0, W, zr_body, 0)
    for k in range(zr // 16):
        zvec[pl.ds(k * 16, 16)] = z16

    for b in range(zr // W):
        pltpu.sync_copy(rows, oacc_sp.at[pl.ds(sid * zr + b * W, W)])
    pltpu.sync_copy(zvec, den_sp.at[pl.ds(sid * zr, zr)])

    descs = []
    for j in range(NWINP):
        descs.append(pltpu.async_copy(
            vi_hbm.at[pl.ds(base + j * W, W)], vbuf.at[j], sem_i))
        descs.append(pltpu.async_copy(
            ei_hbm.at[pl.ds(base + j * W, W)], ebuf.at[j], sem_i))
    for d in descs:
        d.wait()
    plsc.subcore_barrier()

    def win(j, carry):
        pltpu.sync_copy(ys_hbm.at[cid].at[ebuf.at[j]], rows)
        pltpu.sync_copy(ge_hbm.at[ebuf.at[j]], vals)
        pltpu.sync_copy(rows, oacc_sp.at[vbuf.at[j]], add=True)
        pltpu.sync_copy(vals, den_sp.at[vbuf.at[j]], add=True)
        return carry

    lax.fori_loop(0, NWINP, win, 0)
    plsc.subcore_barrier()

    @pl.when(sid < NSUB - 1)
    def _():
        pltpu.sync_copy(oacc_sp.at[pl.ds(sid * BIG, BIG)],
                        osum_hbm.at[cid, pl.ds(sid * BIG, BIG)])

    @pl.when(sid == NSUB - 1)
    def _():
        pltpu.sync_copy(oacc_sp.at[pl.ds((NSUB - 1) * BIG, LAST)],
                        osum_hbm.at[cid, pl.ds((NSUB - 1) * BIG, LAST)])

    @pl.when(cid == 0)
    def _():
        pltpu.sync_copy(den_sp.at[pl.ds(sid * zr, zr)], zvec)
        pltpu.sync_copy(zvec, den_hbm.at[pl.ds(sid * zr, zr)])


def _k4(yscaled2, ge, vp, ep):
    f = pl.kernel(
        _k4_body,
        out_type=(
            jax.ShapeDtypeStruct((2, N, H), jnp.float32),
            jax.ShapeDtypeStruct((NP,), jnp.float32),
        ),
        mesh=_mesh(),
        scratch_types=[
            pltpu.VMEM_SHARED((NP, H), jnp.float32),
            pltpu.VMEM_SHARED((NP,), jnp.float32),
            pltpu.VMEM((NWINP, W), jnp.int32),
            pltpu.VMEM((NWINP, W), jnp.int32),
            pltpu.VMEM((W, H), jnp.float32),
            pltpu.VMEM((W,), jnp.float32),
            pltpu.VMEM((NP // NSUB,), jnp.float32),
            pltpu.SemaphoreType.DMA,
        ],
    )
    return f(yscaled2, ge, vp, ep)


def _k5_body(os_ref, den_ref, out_ref):
    d = jnp.maximum(den_ref[...], 1e-12)
    o = jnp.concatenate([os_ref[0], os_ref[1]], axis=1) / d
    out_ref[...] = jnp.where(o > 0, o, jnp.exp(o) - 1.0)


def _k5(outsum2, den2):
    R = 1000
    return pl.pallas_call(
        _k5_body,
        grid=(N // R,),
        in_specs=[
            pl.BlockSpec((2, R, H), lambda i: (0, i, 0)),
            pl.BlockSpec((R, 1), lambda i: (i, 0)),
        ],
        out_specs=pl.BlockSpec((R, C), lambda i: (i, 0)),
        out_shape=jax.ShapeDtypeStruct((N, C), jnp.float32),
    )(outsum2, den2)


def kernel(X, v_idx, e_idx, W_theta, b_theta, w_atten_e):
    b2 = b_theta.reshape(2, 1, H)
    w2 = w_atten_e.reshape(2, 1, H)
    Xp = jnp.pad(X, ((0, NP - N), (0, 0)))
    pad_i = jnp.arange(NNZP - NNZ, dtype=jnp.int32)
    vp = jnp.concatenate([v_idx, N + pad_i % (NP - N)])
    ep = jnp.concatenate([e_idx, M + pad_i % (MP - M)])
    Xw2 = _k1(Xp, W_theta, b2)
    ysum2, cnt = _k2(Xw2, vp, ep)
    yscaled2, ge = _k3(ysum2, cnt.reshape(MP, 1), w2)
    outsum2, den = _k4(yscaled2, ge.reshape(MP), vp, ep)
    return _k5(outsum2, den[:N].reshape(N, 1))


# speedup vs baseline: 1.7316x; 1.0557x over previous
import jax
import jax.numpy as jnp
from jax import lax
from jax.experimental import pallas as pl
from jax.experimental.pallas import tpu as pltpu
from jax.experimental.pallas import tpu_sc as plsc

N = 10000
M = 5000
NNZ = 160000
C = 256
H = 128
MP = 5120
NP = 10240
NSUB = 16
W = 128
NNZP = 163840
CHUNKP = NNZP // NSUB
NWINP = CHUNKP // W
BIG = 632
LAST = N - (NSUB - 1) * BIG

_mesh = lambda: plsc.VectorSubcoreMesh(core_axis_name="c", subcore_axis_name="s")


def _k1_body(x_ref, w_ref, b_ref, o_ref):
    o_ref[0] = (
        jax.lax.dot_general(
            x_ref[...], w_ref[...], (((1,), (1,)), ((), ())),
            preferred_element_type=jnp.float32,
            precision=jax.lax.Precision.HIGHEST,
        )
        + b_ref[0]
    )


def _k1(Xp, W_theta, b2):
    R = 1024
    return pl.pallas_call(
        _k1_body,
        grid=(NP // R, 2),
        in_specs=[
            pl.BlockSpec((R, C), lambda i, h: (i, 0)),
            pl.BlockSpec((H, C), lambda i, h: (h, 0)),
            pl.BlockSpec((1, 1, H), lambda i, h: (h, 0, 0)),
        ],
        out_specs=pl.BlockSpec((1, R, H), lambda i, h: (h, i, 0)),
        out_shape=jax.ShapeDtypeStruct((2, NP, H), jnp.float32),
    )(Xp, W_theta, b2)


def _k2_body(xw_hbm, vi_hbm, ei_hbm, ysum_hbm, cnt_hbm,
             ysum_sp, cnt_sp, vbuf, ebuf, rows, ones_b, zvec, sem_i):
    cid = lax.axis_index("c")
    sid = lax.axis_index("s")
    zr = MP // NSUB
    base = sid * CHUNKP

    z16 = jnp.zeros((16,), jnp.float32)
    o16 = jnp.ones((16,), jnp.float32)

    def zr_body(i, carry):
        for k in range(H // 16):
            rows[i, pl.ds(k * 16, 16)] = z16
        return carry

    lax.fori_loop(0, W, zr_body, 0)
    for k in range(W // 16):
        ones_b[pl.ds(k * 16, 16)] = o16
    for k in range(zr // 16):
        zvec[pl.ds(k * 16, 16)] = z16

    pltpu.sync_copy(rows, ysum_sp.at[pl.ds(sid * zr, W)])
    pltpu.sync_copy(rows, ysum_sp.at[pl.ds(sid * zr + W, W)])
    pltpu.sync_copy(rows.at[pl.ds(0, zr - 2 * W)],
                    ysum_sp.at[pl.ds(sid * zr + 2 * W, zr - 2 * W)])
    pltpu.sync_copy(zvec, cnt_sp.at[pl.ds(sid * zr, zr)])

    descs = []
    for j in range(NWINP):
        descs.append(pltpu.async_copy(
            vi_hbm.at[pl.ds(base + j * W, W)], vbuf.at[j], sem_i))
        descs.append(pltpu.async_copy(
            ei_hbm.at[pl.ds(base + j * W, W)], ebuf.at[j], sem_i))
    for d in descs:
        d.wait()
    plsc.subcore_barrier()

    def win(j, carry):
        pltpu.sync_copy(xw_hbm.at[cid].at[vbuf.at[j]], rows)
        pltpu.sync_copy(rows, ysum_sp.at[ebuf.at[j]], add=True)
        mine = jnp.logical_or(
            jnp.logical_and(cid == 0, j < NWINP // 2),
            jnp.logical_and(cid == 1, j >= NWINP // 2))

        @pl.when(mine)
        def _():
            pltpu.sync_copy(ones_b, cnt_sp.at[ebuf.at[j]], add=True)

        return carry

    lax.fori_loop(0, NWINP, win, 0)
    plsc.subcore_barrier()

    pltpu.sync_copy(ysum_sp.at[pl.ds(sid * zr, zr)],
                    ysum_hbm.at[cid, pl.ds(sid * zr, zr)])

    pltpu.sync_copy(cnt_sp.at[pl.ds(sid * zr, zr)], zvec)
    pltpu.sync_copy(zvec, cnt_hbm.at[pl.ds(cid * MP + sid * zr, zr)])


def _k2(Xw2, vp, ep):
    f = pl.kernel(
        _k2_body,
        out_type=(
            jax.ShapeDtypeStruct((2, MP, H), jnp.float32),
            jax.ShapeDtypeStruct((2 * MP,), jnp.float32),
        ),
        mesh=_mesh(),
        scratch_types=[
            pltpu.VMEM_SHARED((MP, H), jnp.float32),
            pltpu.VMEM_SHARED((MP,), jnp.float32),
            pltpu.VMEM((NWINP, W), jnp.int32),
            pltpu.VMEM((NWINP, W), jnp.int32),
            pltpu.VMEM((W, H), jnp.float32),
            pltpu.VMEM((W,), jnp.float32),
            pltpu.VMEM((MP // NSUB,), jnp.float32),
            pltpu.SemaphoreType.DMA,
        ],
    )
    return f(Xw2, vp, ep)


def _k3_body(ysum_ref, cnt_ref, w_ref, ys_ref, ge_ref):
    c = jnp.maximum(cnt_ref[0] + cnt_ref[1], 1.0)
    y0 = ysum_ref[0] / c
    y1 = ysum_ref[1] / c
    a = jnp.sum(y0 * w_ref[0] + y1 * w_ref[1], axis=1, keepdims=True)
    a = jnp.where(a >= 0, a, 0.2 * a)
    a = jnp.clip(a, 0.001, 5.0)
    g = jnp.exp(a)
    ys_ref[0] = g * y0
    ys_ref[1] = g * y1
    ge_ref[...] = g


def _k3(ysum2, cnt2, w2):
    R = 640
    return pl.pallas_call(
        _k3_body,
        grid=(MP // R,),
        in_specs=[
            pl.BlockSpec((2, R, H), lambda i: (0, i, 0)),
            pl.BlockSpec((2, R, 1), lambda i: (0, i, 0)),
            pl.BlockSpec((2, 1, H), lambda i: (0, 0, 0)),
        ],
        out_specs=[
            pl.BlockSpec((2, R, H), lambda i: (0, i, 0)),
            pl.BlockSpec((R, 1), lambda i: (i, 0)),
        ],
        out_shape=[
            jax.ShapeDtypeStruct((2, MP, H), jnp.float32),
            jax.ShapeDtypeStruct((MP, 1), jnp.float32),
        ],
    )(ysum2, cnt2, w2)


def _k4_body(ys_hbm, ge_hbm, vi_hbm, ei_hbm, osum_hbm, den_hbm,
             oacc_sp, den_sp, vbuf, ebuf, rows, vals, zvec, sem_i):
    cid = lax.axis_index("c")
    sid = lax.axis_index("s")
    zr = NP // NSUB
    base = sid * CHUNKP

    z16 = jnp.zeros((16,), jnp.float32)

    def zr_body(i, carry):
        for k in range(H // 16):
            rows[i, pl.ds(k * 16, 16)] = z16
        return carry

    lax.fori_loop(0, W, zr_body, 0)
    for k in range(zr // 16):
        zvec[pl.ds(k * 16, 16)] = z16

    for b in range(zr // W):
        pltpu.sync_copy(rows, oacc_sp.at[pl.ds(sid * zr + b * W, W)])
    pltpu.sync_copy(zvec, den_sp.at[pl.ds(sid * zr, zr)])

    descs = []
    for j in range(NWINP):
        descs.append(pltpu.async_copy(
            vi_hbm.at[pl.ds(base + j * W, W)], vbuf.at[j], sem_i))
        descs.append(pltpu.async_copy(
            ei_hbm.at[pl.ds(base + j * W, W)], ebuf.at[j], sem_i))
    for d in descs:
        d.wait()
    plsc.subcore_barrier()

    def win(j, carry):
        pltpu.sync_copy(ys_hbm.at[cid].at[ebuf.at[j]], rows)
        mine = jnp.logical_or(
            jnp.logical_and(cid == 0, j < NWINP // 2),
            jnp.logical_and(cid == 1, j >= NWINP // 2))

        @pl.when(mine)
        def _():
            pltpu.sync_copy(ge_hbm.at[ebuf.at[j]], vals)

        pltpu.sync_copy(rows, oacc_sp.at[vbuf.at[j]], add=True)

        @pl.when(mine)
        def _():
            pltpu.sync_copy(vals, den_sp.at[vbuf.at[j]], add=True)

        return carry

    lax.fori_loop(0, NWINP, win, 0)
    plsc.subcore_barrier()

    @pl.when(sid < NSUB - 1)
    def _():
        pltpu.sync_copy(oacc_sp.at[pl.ds(sid * BIG, BIG)],
                        osum_hbm.at[cid, pl.ds(sid * BIG, BIG)])

    @pl.when(sid == NSUB - 1)
    def _():
        pltpu.sync_copy(oacc_sp.at[pl.ds((NSUB - 1) * BIG, LAST)],
                        osum_hbm.at[cid, pl.ds((NSUB - 1) * BIG, LAST)])

    pltpu.sync_copy(den_sp.at[pl.ds(sid * zr, zr)], zvec)
    pltpu.sync_copy(zvec, den_hbm.at[pl.ds(cid * NP + sid * zr, zr)])


def _k4(yscaled2, ge, vp, ep):
    f = pl.kernel(
        _k4_body,
        out_type=(
            jax.ShapeDtypeStruct((2, N, H), jnp.float32),
            jax.ShapeDtypeStruct((2 * NP,), jnp.float32),
        ),
        mesh=_mesh(),
        scratch_types=[
            pltpu.VMEM_SHARED((NP, H), jnp.float32),
            pltpu.VMEM_SHARED((NP,), jnp.float32),
            pltpu.VMEM((NWINP, W), jnp.int32),
            pltpu.VMEM((NWINP, W), jnp.int32),
            pltpu.VMEM((W, H), jnp.float32),
            pltpu.VMEM((W,), jnp.float32),
            pltpu.VMEM((NP // NSUB,), jnp.float32),
            pltpu.SemaphoreType.DMA,
        ],
    )
    return f(yscaled2, ge, vp, ep)


def _k5_body(os_ref, den_ref, out_ref):
    d = jnp.maximum(den_ref[0] + den_ref[1], 1e-12)
    o = jnp.concatenate([os_ref[0], os_ref[1]], axis=1) / d
    out_ref[...] = jnp.where(o > 0, o, jnp.exp(o) - 1.0)


def _k5(outsum2, den2):
    R = 1000
    return pl.pallas_call(
        _k5_body,
        grid=(N // R,),
        in_specs=[
            pl.BlockSpec((2, R, H), lambda i: (0, i, 0)),
            pl.BlockSpec((2, R, 1), lambda i: (0, i, 0)),
        ],
        out_specs=pl.BlockSpec((R, C), lambda i: (i, 0)),
        out_shape=jax.ShapeDtypeStruct((N, C), jnp.float32),
    )(outsum2, den2)


def kernel(X, v_idx, e_idx, W_theta, b_theta, w_atten_e):
    b2 = b_theta.reshape(2, 1, H)
    w2 = w_atten_e.reshape(2, 1, H)
    Xp = jnp.pad(X, ((0, NP - N), (0, 0)))
    pad_i = jnp.arange(NNZP - NNZ, dtype=jnp.int32)
    vp = jnp.concatenate([v_idx, N + pad_i % (NP - N)])
    ep = jnp.concatenate([e_idx, M + pad_i % (MP - M)])
    Xw2 = _k1(Xp, W_theta, b2)
    ysum2, cnt = _k2(Xw2, vp, ep)
    yscaled2, ge = _k3(ysum2, cnt.reshape(2, MP, 1), w2)
    outsum2, den = _k4(yscaled2, ge.reshape(MP), vp, ep)
    return _k5(outsum2, den.reshape(2, NP, 1))
